# Initial kernel scaffold; baseline (speedup 1.0000x reference)
import functools

import jax
import jax.numpy as jnp
from jax import lax
from jax.experimental import pallas as pl
from jax.experimental.pallas import tpu as pltpu
from jax.experimental.pallas import tpu_sc as plsc

N = 10000
D = 128
NEG_SLOP = 0.2

NC = 2
NS = 16
L = 16
NW = NC * NS
B = 128
NPT = N // NS
RB = 1000

_MESH = plsc.VectorSubcoreMesh(
    core_axis_name="c", subcore_axis_name="s", num_cores=NC, num_subcores=NS)

_HIGH = lax.Precision.HIGHEST


def _leaky(v):
    return jnp.where(v >= 0, v, NEG_SLOP * v)


def _edge0_body(nblk, hlin, src, dst, ew, znd, zdg, mp, dp,
                srcv, dstv, ewv, rows, wrow, acc, dacc, sem):
    c = lax.axis_index("c")
    s = lax.axis_index("s")
    wid = s * NC + c
    base_n = s * NPT
    pltpu.sync_copy(znd.at[pl.ds(base_n, NPT)], acc.at[pl.ds(base_n, NPT)])
    pltpu.sync_copy(zdg.at[pl.ds(base_n, NPT)], dacc.at[pl.ds(base_n, NPT)])
    plsc.subcore_barrier()

    ebase = wid * (nblk * B)

    def blk(g, carry):
        off = pl.multiple_of(ebase + g * B, B)
        pltpu.sync_copy(src.at[pl.ds(off, B)], srcv)
        pltpu.sync_copy(dst.at[pl.ds(off, B)], dstv)
        pltpu.sync_copy(ew.at[pl.ds(off, B)], ewv)
        pltpu.async_copy(hlin.at[srcv], rows, sem).wait()

        def srow(i, cc):
            cvec = plsc.load_gather(ewv, [jnp.full((L,), i, jnp.int32)])
            wrow[i, :] = cvec
            for j in range(D // L):
                rows[i, pl.ds(j * L, L)] = rows[i, pl.ds(j * L, L)] * cvec
            return cc
        lax.fori_loop(0, B, srow, 0)

        pltpu.sync_copy(rows, acc.at[dstv], add=True)
        pltpu.sync_copy(wrow, dacc.at[dstv], add=True)
        return carry

    lax.fori_loop(0, nblk, blk, 0)
    plsc.subcore_barrier()
    pltpu.sync_copy(acc.at[pl.ds(base_n, NPT)], mp.at[c, pl.ds(base_n, NPT)])
    pltpu.sync_copy(dacc.at[pl.ds(base_n, NPT)], dp.at[c, pl.ds(base_n, NPT)])


def _edge_norm_body(nblk, hlin, src, dst, ew, dinv, znd, mp,
                    srcv, dstv, ewv, coefv, dinvv, rows, acc, sem):
    c = lax.axis_index("c")
    s = lax.axis_index("s")
    wid = s * NC + c
    base_n = s * NPT
    pltpu.sync_copy(dinv, dinvv)
    pltpu.sync_copy(znd.at[pl.ds(base_n, NPT)], acc.at[pl.ds(base_n, NPT)])
    plsc.subcore_barrier()

    ebase = wid * (nblk * B)

    def blk(g, carry):
        off = pl.multiple_of(ebase + g * B, B)
        pltpu.sync_copy(src.at[pl.ds(off, B)], srcv)
        pltpu.sync_copy(dst.at[pl.ds(off, B)], dstv)
        pltpu.sync_copy(ew.at[pl.ds(off, B)], ewv)
        pltpu.async_copy(hlin.at[srcv], rows, sem).wait()

        for grp in range(B // L):
            s16 = srcv[pl.ds(grp * L, L)]
            d16 = dstv[pl.ds(grp * L, L)]
            w16 = ewv[pl.ds(grp * L, L)]
            cv = plsc.load_gather(dinvv, [s16]) * w16 \
                * plsc.load_gather(dinvv, [d16])
            coefv[pl.ds(grp * L, L)] = cv

        def srow(i, cc):
            cvec = plsc.load_gather(coefv, [jnp.full((L,), i, jnp.int32)])
            for j in range(D // L):
                rows[i, pl.ds(j * L, L)] = rows[i, pl.ds(j * L, L)] * cvec
            return cc
        lax.fori_loop(0, B, srow, 0)

        pltpu.sync_copy(rows, acc.at[dstv], add=True)
        return carry

    lax.fori_loop(0, nblk, blk, 0)
    plsc.subcore_barrier()
    pltpu.sync_copy(acc.at[pl.ds(base_n, NPT)], mp.at[c, pl.ds(base_n, NPT)])


def _edge0_call(nblk, hlin, src, dst, ew, znd, zdg):
    return pl.kernel(
        functools.partial(_edge0_body, nblk),
        out_type=[jax.ShapeDtypeStruct((NC, N, D), jnp.float32),
                  jax.ShapeDtypeStruct((NC, N, L), jnp.float32)],
        mesh=_MESH,
        scratch_types=[
            pltpu.VMEM((B,), jnp.int32),
            pltpu.VMEM((B,), jnp.int32),
            pltpu.VMEM((B,), jnp.float32),
            pltpu.VMEM((B, D), jnp.float32),
            pltpu.VMEM((B, L), jnp.float32),
            pltpu.VMEM_SHARED((N, D), jnp.float32),
            pltpu.VMEM_SHARED((N, L), jnp.float32),
            pltpu.SemaphoreType.DMA,
        ],
    )(hlin, src, dst, ew, znd, zdg)


def _edge_norm_call(nblk, hlin, src, dst, ew, dinv, znd):
    return pl.kernel(
        functools.partial(_edge_norm_body, nblk),
        out_type=jax.ShapeDtypeStruct((NC, N, D), jnp.float32),
        mesh=_MESH,
        scratch_types=[
            pltpu.VMEM((B,), jnp.int32),
            pltpu.VMEM((B,), jnp.int32),
            pltpu.VMEM((B,), jnp.float32),
            pltpu.VMEM((B,), jnp.float32),
            pltpu.VMEM((N,), jnp.float32),
            pltpu.VMEM((B, D), jnp.float32),
            pltpu.VMEM_SHARED((N, D), jnp.float32),
            pltpu.SemaphoreType.DMA,
        ],
    )(hlin, src, dst, ew, dinv, znd)


def _row_spec(shape_tail):
    nt = len(shape_tail)
    return pl.BlockSpec((RB,) + shape_tail, lambda i, _nt=nt: (i,) + (0,) * _nt)


def _full_spec(shape):
    nd = len(shape)
    return pl.BlockSpec(shape, lambda i, _nd=nd: (0,) * _nd)


def _tc1_body(x, w, o):
    o[...] = jnp.dot(x[...], w[...], preferred_element_type=jnp.float32,
                     precision=_HIGH)


def _tc2_body(mp, dp, b0, w1, h1_o, hlin1_o, dinv_o):
    m = mp[0] + mp[1] + b0[...][None, :]
    h1 = _leaky(m)
    h1_o[...] = h1
    hlin1_o[...] = jnp.dot(h1, w1[...], preferred_element_type=jnp.float32,
                           precision=_HIGH)
    deg = jnp.max(dp[0] + dp[1], axis=1) + 1.0
    dinv_o[...] = lax.rsqrt(deg)


def _tc3_body(mp, hlin, dinv, b, w, h_o, hlin_o):
    selfc = dinv[...] * dinv[...]
    m = mp[0] + mp[1] + hlin[...] * selfc[:, None] + b[...][None, :]
    h = _leaky(m)
    h_o[...] = h
    hlin_o[...] = jnp.dot(h, w[...], preferred_element_type=jnp.float32,
                          precision=_HIGH)


def _tc4_body(mp, hlin, dinv, b, h1, h2, wh, bh, o):
    selfc = dinv[...] * dinv[...]
    m = mp[0] + mp[1] + hlin[...] * selfc[:, None] + b[...][None, :]
    h3 = _leaky(m)
    acc = jnp.dot(h1[...], wh[0:D, :], preferred_element_type=jnp.float32,
                  precision=_HIGH)
    acc += jnp.dot(h2[...], wh[D:2 * D, :], preferred_element_type=jnp.float32,
                   precision=_HIGH)
    acc += jnp.dot(h3, wh[2 * D:3 * D, :], preferred_element_type=jnp.float32,
                   precision=_HIGH)
    o[...] = acc + bh[...][None, :]


_GRID = N // RB
_F32 = jnp.float32


def _sds(shape):
    return jax.ShapeDtypeStruct(shape, _F32)


def _parts_spec(lanes):
    return pl.BlockSpec((NC, RB, lanes), lambda i: (0, i, 0))


def kernel(x, edge_index, edge_weight, W0, b0, W1, b1, W2, b2, Wh, bh):
    E = edge_index.shape[1]
    epw = -(-E // (NW * B)) * B
    ep = epw * NW
    pad = ep - E
    src = edge_index[0].astype(jnp.int32)
    dst = edge_index[1].astype(jnp.int32)
    ew = edge_weight.astype(jnp.float32)
    if pad:
        zi = jnp.zeros((pad,), jnp.int32)
        src = jnp.concatenate([src, zi])
        dst = jnp.concatenate([dst, zi])
        ew = jnp.concatenate([ew, jnp.zeros((pad,), jnp.float32)])
    znd = jnp.zeros((N, D), jnp.float32)
    zdg = jnp.zeros((N, L), jnp.float32)
    nblk = epw // B

    hlin0 = pl.pallas_call(
        _tc1_body, grid=(_GRID,),
        in_specs=[_row_spec((D,)), _full_spec((D, D))],
        out_specs=_row_spec((D,)),
        out_shape=_sds((N, D)))(x, W0)

    m0p, degp = _edge0_call(nblk, hlin0, src, dst, ew, znd, zdg)

    h1, hlin1, dinv = pl.pallas_call(
        _tc2_body, grid=(_GRID,),
        in_specs=[_parts_spec(D), _parts_spec(L), _full_spec((D,)),
                  _full_spec((D, D))],
        out_specs=[_row_spec(()), _row_spec(()), None][0:0] or
                  [_row_spec((D,)), _row_spec((D,)),
                   pl.BlockSpec((RB,), lambda i: (i,))],
        out_shape=[_sds((N, D)), _sds((N, D)), _sds((N,))],
    )(m0p, degp, b0, W1)

    m1p = _edge_norm_call(nblk, hlin1, src, dst, ew, dinv, znd)

    h2, hlin2 = pl.pallas_call(
        _tc3_body, grid=(_GRID,),
        in_specs=[_parts_spec(D), _row_spec((D,)),
                  pl.BlockSpec((RB,), lambda i: (i,)), _full_spec((D,)),
                  _full_spec((D, D))],
        out_specs=[_row_spec((D,)), _row_spec((D,))],
        out_shape=[_sds((N, D)), _sds((N, D))],
    )(m1p, hlin1, dinv, b1, W2)

    m2p = _edge_norm_call(nblk, hlin2, src, dst, ew, dinv, znd)

    out = pl.pallas_call(
        _tc4_body, grid=(_GRID,),
        in_specs=[_parts_spec(D), _row_spec((D,)),
                  pl.BlockSpec((RB,), lambda i: (i,)), _full_spec((D,)),
                  _row_spec((D,)), _row_spec((D,)),
                  _full_spec((3 * D, D)), _full_spec((D,))],
        out_specs=_row_spec((D,)),
        out_shape=_sds((N, D)),
    )(m2p, hlin2, dinv, b2, h1, h2, Wh, bh)
    return out


# speedup vs baseline: 7.0798x; 7.0798x over previous
import functools

import jax
import jax.numpy as jnp
from jax import lax
from jax.experimental import pallas as pl
from jax.experimental.pallas import tpu as pltpu
from jax.experimental.pallas import tpu_sc as plsc

N = 10000
D = 128
NEG_SLOP = 0.2

NC = 2
NS = 16
L = 16
NW = NC * NS
B = 128
NPT = 640
NP = NS * NPT
RB = 1024

_MESH = plsc.VectorSubcoreMesh(
    core_axis_name="c", subcore_axis_name="s", num_cores=NC, num_subcores=NS)
_SC_PARAMS = pltpu.CompilerParams(needs_layout_passes=False)
_HIGH = lax.Precision.HIGHEST


def _leaky(v):
    return jnp.where(v >= 0, v, NEG_SLOP * v)


def _scale_rows(rows, coefv):
    def srow(i, cc):
        cvec = plsc.load_gather(coefv, [jnp.full((L,), i, jnp.int32)])
        for j in range(D // L):
            rows[i, pl.ds(j * L, L)] = rows[i, pl.ds(j * L, L)] * cvec
        return cc
    lax.fori_loop(0, B, srow, 0)


def _edge0_body(nblk, hlin, src, dst, ew, znd, mp, dp,
                srcv, dstv, ewv, rows, degl, acc, sem):
    c = lax.axis_index("c")
    s = lax.axis_index("s")
    wid = s * NC + c
    base_n = s * NPT
    pltpu.sync_copy(znd.at[pl.ds(base_n, NPT)], acc.at[pl.ds(base_n, NPT)])
    z16 = jnp.zeros((L,), jnp.float32)

    def zz(i, cc):
        degl[pl.ds(i * L, L)] = z16
        return cc
    lax.fori_loop(0, NP // L, zz, 0)
    plsc.subcore_barrier()

    ebase = wid * (nblk * B)

    def blk(g, carry):
        off = pl.multiple_of(ebase + g * B, B)
        pltpu.sync_copy(src.at[pl.ds(off, B)], srcv)
        pltpu.sync_copy(dst.at[pl.ds(off, B)], dstv)
        pltpu.sync_copy(ew.at[pl.ds(off, B)], ewv)
        pltpu.async_copy(hlin.at[srcv], rows, sem).wait()
        for grp in range(B // L):
            d16 = dstv[pl.ds(grp * L, L)]
            w16 = ewv[pl.ds(grp * L, L)]
            plsc.addupdate_scatter(degl, [d16], w16)
        _scale_rows(rows, ewv)
        pltpu.sync_copy(rows, acc.at[dstv], add=True)
        return carry

    lax.fori_loop(0, nblk, blk, 0)
    plsc.subcore_barrier()
    pltpu.sync_copy(acc.at[pl.ds(base_n, NPT)], mp.at[c, pl.ds(base_n, NPT)])
    pltpu.sync_copy(degl, dp.at[pl.ds(wid * NP, NP)])


def _edge_norm_body(nblk, hlin, src, dst, ew, dinv, znd, mp,
                    srcv, dstv, ewv, coefv, dinvv, rows, acc, sem):
    c = lax.axis_index("c")
    s = lax.axis_index("s")
    wid = s * NC + c
    base_n = s * NPT
    pltpu.sync_copy(dinv, dinvv)
    pltpu.sync_copy(znd.at[pl.ds(base_n, NPT)], acc.at[pl.ds(base_n, NPT)])
    plsc.subcore_barrier()

    ebase = wid * (nblk * B)

    def blk(g, carry):
        off = pl.multiple_of(ebase + g * B, B)
        pltpu.sync_copy(src.at[pl.ds(off, B)], srcv)
        pltpu.sync_copy(dst.at[pl.ds(off, B)], dstv)
        pltpu.sync_copy(ew.at[pl.ds(off, B)], ewv)
        pltpu.async_copy(hlin.at[srcv], rows, sem).wait()
        for grp in range(B // L):
            s16 = srcv[pl.ds(grp * L, L)]
            d16 = dstv[pl.ds(grp * L, L)]
            w16 = ewv[pl.ds(grp * L, L)]
            cv = plsc.load_gather(dinvv, [s16]) * w16 \
                * plsc.load_gather(dinvv, [d16])
            coefv[pl.ds(grp * L, L)] = cv
        _scale_rows(rows, coefv)
        pltpu.sync_copy(rows, acc.at[dstv], add=True)
        return carry

    lax.fori_loop(0, nblk, blk, 0)
    plsc.subcore_barrier()
    pltpu.sync_copy(acc.at[pl.ds(base_n, NPT)], mp.at[c, pl.ds(base_n, NPT)])


def _edge0_call(nblk, hlin, src, dst, ew, znd):
    return pl.kernel(
        functools.partial(_edge0_body, nblk),
        out_type=[jax.ShapeDtypeStruct((NC, NP, D), jnp.float32),
                  jax.ShapeDtypeStruct((NW * NP,), jnp.float32)],
        mesh=_MESH,
        compiler_params=_SC_PARAMS,
        scratch_types=[
            pltpu.VMEM((B,), jnp.int32),
            pltpu.VMEM((B,), jnp.int32),
            pltpu.VMEM((B,), jnp.float32),
            pltpu.VMEM((B, D), jnp.float32),
            pltpu.VMEM((NP,), jnp.float32),
            pltpu.VMEM_SHARED((NP, D), jnp.float32),
            pltpu.SemaphoreType.DMA,
        ],
    )(hlin, src, dst, ew, znd)


def _edge_norm_call(nblk, hlin, src, dst, ew, dinv, znd):
    return pl.kernel(
        functools.partial(_edge_norm_body, nblk),
        out_type=jax.ShapeDtypeStruct((NC, NP, D), jnp.float32),
        mesh=_MESH,
        compiler_params=_SC_PARAMS,
        scratch_types=[
            pltpu.VMEM((B,), jnp.int32),
            pltpu.VMEM((B,), jnp.int32),
            pltpu.VMEM((B,), jnp.float32),
            pltpu.VMEM((B,), jnp.float32),
            pltpu.VMEM((NP,), jnp.float32),
            pltpu.VMEM((B, D), jnp.float32),
            pltpu.VMEM_SHARED((NP, D), jnp.float32),
            pltpu.SemaphoreType.DMA,
        ],
    )(hlin, src, dst, ew, dinv, znd)


def _row_spec(shape_tail):
    nt = len(shape_tail)
    return pl.BlockSpec((RB,) + shape_tail, lambda i, _nt=nt: (i,) + (0,) * _nt)


def _full_spec(shape):
    nd = len(shape)
    return pl.BlockSpec(shape, lambda i, _nd=nd: (0,) * _nd)


def _parts_spec():
    return pl.BlockSpec((NC, RB, D), lambda i: (0, i, 0))


def _tc_mm_body(x, w, o):
    o[...] = jnp.dot(x[...], w[...], preferred_element_type=jnp.float32,
                     precision=_HIGH)


def _tc_deg_body(dp, dinv_o):
    deg = jnp.sum(dp[...], axis=0) + 1.0
    dinv_o[...] = lax.rsqrt(deg)


def _tc_comb_body(mp, b, w, h_o, hlin_o):
    m = mp[0] + mp[1] + b[...][None, :]
    h = _leaky(m)
    h_o[...] = h
    hlin_o[...] = jnp.dot(h, w[...], preferred_element_type=jnp.float32,
                          precision=_HIGH)


def _tc_final_body(mp, b, h1, h2, wh, bh, o):
    m = mp[0] + mp[1] + b[...][None, :]
    h3 = _leaky(m)
    acc = jnp.dot(h1[...], wh[0:D, :], preferred_element_type=jnp.float32,
                  precision=_HIGH)
    acc += jnp.dot(h2[...], wh[D:2 * D, :], preferred_element_type=jnp.float32,
                   precision=_HIGH)
    acc += jnp.dot(h3, wh[2 * D:3 * D, :], preferred_element_type=jnp.float32,
                   precision=_HIGH)
    o[...] = acc + bh[...][None, :]


_GRID = NP // RB
_F32 = jnp.float32


def _sds(shape):
    return jax.ShapeDtypeStruct(shape, _F32)


def _pad1(v, tot, dtype):
    return jnp.concatenate([v, jnp.zeros((tot - v.shape[0],), dtype)])


def kernel(x, edge_index, edge_weight, W0, b0, W1, b1, W2, b2, Wh, bh):
    E = edge_index.shape[1]
    src = edge_index[0].astype(jnp.int32)
    dst = edge_index[1].astype(jnp.int32)
    ew = edge_weight.astype(jnp.float32)

    epw0 = -(-E // (NW * B)) * B
    nblk0 = epw0 // B
    src0 = _pad1(src, NW * epw0, jnp.int32)
    dst0 = _pad1(dst, NW * epw0, jnp.int32)
    ew0 = _pad1(ew, NW * epw0, jnp.float32)

    loop = jnp.arange(N, dtype=jnp.int32)
    E1 = E + N
    epw1 = -(-E1 // (NW * B)) * B
    nblk1 = epw1 // B
    src1 = _pad1(jnp.concatenate([src, loop]), NW * epw1, jnp.int32)
    dst1 = _pad1(jnp.concatenate([dst, loop]), NW * epw1, jnp.int32)
    ew1 = _pad1(jnp.concatenate([ew, jnp.ones((N,), jnp.float32)]),
                NW * epw1, jnp.float32)

    xp = jnp.concatenate([x, jnp.zeros((NP - N, x.shape[1]), jnp.float32)])
    znd = jnp.zeros((NP, D), jnp.float32)

    mm = pl.pallas_call(
        _tc_mm_body, grid=(_GRID,),
        in_specs=[_row_spec((D,)), _full_spec((D, D))],
        out_specs=_row_spec((D,)),
        out_shape=_sds((NP, D)))

    hlin0 = mm(xp, W0)
    m0p, degp = _edge0_call(nblk0, hlin0, src0, dst0, ew0, znd)

    dinv = pl.pallas_call(
        _tc_deg_body, grid=(1,),
        in_specs=[pl.BlockSpec((NW, NP), lambda i: (0, 0))],
        out_specs=_full_spec((NP,)),
        out_shape=_sds((NP,)))(degp.reshape(NW, NP))

    comb = pl.pallas_call(
        _tc_comb_body, grid=(_GRID,),
        in_specs=[_parts_spec(), _full_spec((D,)), _full_spec((D, D))],
        out_specs=[_row_spec((D,)), _row_spec((D,))],
        out_shape=[_sds((NP, D)), _sds((NP, D))])

    h1, hlin1 = comb(m0p, b0, W1)
    m1p = _edge_norm_call(nblk1, hlin1, src1, dst1, ew1, dinv, znd)
    h2, hlin2 = comb(m1p, b1, W2)
    m2p = _edge_norm_call(nblk1, hlin2, src1, dst1, ew1, dinv, znd)

    out = pl.pallas_call(
        _tc_final_body, grid=(_GRID,),
        in_specs=[_parts_spec(), _full_spec((D,)),
                  _row_spec((D,)), _row_spec((D,)),
                  _full_spec((3 * D, D)), _full_spec((D,))],
        out_specs=_row_spec((D,)),
        out_shape=_sds((NP, D)),
    )(m2p, b2, h1, h2, Wh, bh)
    return out[0:N]


# speedup vs baseline: 7.8172x; 1.1042x over previous
import functools

import jax
import jax.numpy as jnp
from jax import lax
from jax.experimental import pallas as pl
from jax.experimental.pallas import tpu as pltpu
from jax.experimental.pallas import tpu_sc as plsc

N = 10000
D = 128
NEG_SLOP = 0.2

NC = 2
NS = 16
L = 16
NW = NC * NS
B = 128
NPT = 640
NP = NS * NPT
RB = 1024

_MESH = plsc.VectorSubcoreMesh(
    core_axis_name="c", subcore_axis_name="s", num_cores=NC, num_subcores=NS)
_SC_PARAMS = pltpu.CompilerParams(needs_layout_passes=False)
_HIGH = lax.Precision.HIGHEST


def _leaky(v):
    return jnp.where(v >= 0, v, NEG_SLOP * v)


def _scale_rows(rows, coefv):
    @plsc.parallel_loop(0, B, 1, unroll=4)
    def _(i):
        cvec = plsc.load_gather(coefv, [jnp.full((L,), i, jnp.int32)])
        for j in range(D // L):
            rows[i, pl.ds(j * L, L)] = rows[i, pl.ds(j * L, L)] * cvec


def _edge0_body(nblk, hlin, src, dst, ew, znd, mp, dp,
                srcv, dstv, ewv, rows, degl, acc, sem):
    c = lax.axis_index("c")
    s = lax.axis_index("s")
    wid = s * NC + c
    base_n = s * NPT
    pltpu.sync_copy(znd.at[pl.ds(base_n, NPT)], acc.at[pl.ds(base_n, NPT)])
    z16 = jnp.zeros((L,), jnp.float32)

    def zz(i, cc):
        degl[pl.ds(i * L, L)] = z16
        return cc
    lax.fori_loop(0, NP // L, zz, 0)
    plsc.subcore_barrier()

    ebase = wid * (nblk * B)

    def blk(g, carry):
        off = pl.multiple_of(ebase + g * B, B)
        pltpu.sync_copy(src.at[pl.ds(off, B)], srcv)
        pltpu.sync_copy(dst.at[pl.ds(off, B)], dstv)
        pltpu.sync_copy(ew.at[pl.ds(off, B)], ewv)
        pltpu.async_copy(hlin.at[srcv], rows, sem).wait()
        for grp in range(B // L):
            d16 = dstv[pl.ds(grp * L, L)]
            w16 = ewv[pl.ds(grp * L, L)]
            plsc.addupdate_scatter(degl, [d16], w16)
        _scale_rows(rows, ewv)
        pltpu.sync_copy(rows, acc.at[dstv], add=True)
        return carry

    lax.fori_loop(0, nblk, blk, 0)
    plsc.subcore_barrier()
    pltpu.sync_copy(acc.at[pl.ds(base_n, NPT)], mp.at[c, pl.ds(base_n, NPT)])
    pltpu.sync_copy(degl, dp.at[pl.ds(wid * NP, NP)])


def _edge_norm_body(nblk, hlin, src, dst, ew, dinv, znd, mp,
                    srcv, dstv, ewv, coefv, dinvv, rows, acc, sem):
    c = lax.axis_index("c")
    s = lax.axis_index("s")
    wid = s * NC + c
    base_n = s * NPT
    pltpu.sync_copy(dinv, dinvv)
    pltpu.sync_copy(znd.at[pl.ds(base_n, NPT)], acc.at[pl.ds(base_n, NPT)])
    plsc.subcore_barrier()

    ebase = wid * (nblk * B)

    def blk(g, carry):
        off = pl.multiple_of(ebase + g * B, B)
        pltpu.sync_copy(src.at[pl.ds(off, B)], srcv)
        pltpu.sync_copy(dst.at[pl.ds(off, B)], dstv)
        pltpu.sync_copy(ew.at[pl.ds(off, B)], ewv)
        pltpu.async_copy(hlin.at[srcv], rows, sem).wait()
        for grp in range(B // L):
            s16 = srcv[pl.ds(grp * L, L)]
            d16 = dstv[pl.ds(grp * L, L)]
            w16 = ewv[pl.ds(grp * L, L)]
            cv = plsc.load_gather(dinvv, [s16]) * w16 \
                * plsc.load_gather(dinvv, [d16])
            coefv[pl.ds(grp * L, L)] = cv
        _scale_rows(rows, coefv)
        pltpu.sync_copy(rows, acc.at[dstv], add=True)
        return carry

    lax.fori_loop(0, nblk, blk, 0)
    plsc.subcore_barrier()
    pltpu.sync_copy(acc.at[pl.ds(base_n, NPT)], mp.at[c, pl.ds(base_n, NPT)])


def _edge0_call(nblk, hlin, src, dst, ew, znd):
    return pl.kernel(
        functools.partial(_edge0_body, nblk),
        out_type=[jax.ShapeDtypeStruct((NC, NP, D), jnp.float32),
                  jax.ShapeDtypeStruct((NW * NP,), jnp.float32)],
        mesh=_MESH,
        compiler_params=_SC_PARAMS,
        scratch_types=[
            pltpu.VMEM((B,), jnp.int32),
            pltpu.VMEM((B,), jnp.int32),
            pltpu.VMEM((B,), jnp.float32),
            pltpu.VMEM((B, D), jnp.float32),
            pltpu.VMEM((NP,), jnp.float32),
            pltpu.VMEM_SHARED((NP, D), jnp.float32),
            pltpu.SemaphoreType.DMA,
        ],
    )(hlin, src, dst, ew, znd)


def _edge_norm_call(nblk, hlin, src, dst, ew, dinv, znd):
    return pl.kernel(
        functools.partial(_edge_norm_body, nblk),
        out_type=jax.ShapeDtypeStruct((NC, NP, D), jnp.float32),
        mesh=_MESH,
        compiler_params=_SC_PARAMS,
        scratch_types=[
            pltpu.VMEM((B,), jnp.int32),
            pltpu.VMEM((B,), jnp.int32),
            pltpu.VMEM((B,), jnp.float32),
            pltpu.VMEM((B,), jnp.float32),
            pltpu.VMEM((NP,), jnp.float32),
            pltpu.VMEM((B, D), jnp.float32),
            pltpu.VMEM_SHARED((NP, D), jnp.float32),
            pltpu.SemaphoreType.DMA,
        ],
    )(hlin, src, dst, ew, dinv, znd)


def _row_spec(shape_tail):
    nt = len(shape_tail)
    return pl.BlockSpec((RB,) + shape_tail, lambda i, _nt=nt: (i,) + (0,) * _nt)


def _full_spec(shape):
    nd = len(shape)
    return pl.BlockSpec(shape, lambda i, _nd=nd: (0,) * _nd)


def _parts_spec():
    return pl.BlockSpec((NC, RB, D), lambda i: (0, i, 0))


def _tc_mm_body(x, w, o):
    o[...] = jnp.dot(x[...], w[...], preferred_element_type=jnp.float32,
                     precision=_HIGH)


def _tc_deg_body(dp, dinv_o):
    deg = jnp.sum(dp[...], axis=0) + 1.0
    dinv_o[...] = lax.rsqrt(deg)


def _tc_comb_body(mp, b, w, h_o, hlin_o):
    m = mp[0] + mp[1] + b[...][None, :]
    h = _leaky(m)
    h_o[...] = h
    hlin_o[...] = jnp.dot(h, w[...], preferred_element_type=jnp.float32,
                          precision=_HIGH)


def _tc_final_body(mp, b, h1, h2, wh, bh, o):
    m = mp[0] + mp[1] + b[...][None, :]
    h3 = _leaky(m)
    acc = jnp.dot(h1[...], wh[0:D, :], preferred_element_type=jnp.float32,
                  precision=_HIGH)
    acc += jnp.dot(h2[...], wh[D:2 * D, :], preferred_element_type=jnp.float32,
                   precision=_HIGH)
    acc += jnp.dot(h3, wh[2 * D:3 * D, :], preferred_element_type=jnp.float32,
                   precision=_HIGH)
    o[...] = acc + bh[...][None, :]


_GRID = NP // RB
_F32 = jnp.float32


def _sds(shape):
    return jax.ShapeDtypeStruct(shape, _F32)


def _pad1(v, tot, dtype):
    return jnp.concatenate([v, jnp.zeros((tot - v.shape[0],), dtype)])


def kernel(x, edge_index, edge_weight, W0, b0, W1, b1, W2, b2, Wh, bh):
    E = edge_index.shape[1]
    src = edge_index[0].astype(jnp.int32)
    dst = edge_index[1].astype(jnp.int32)
    ew = edge_weight.astype(jnp.float32)

    epw0 = -(-E // (NW * B)) * B
    nblk0 = epw0 // B
    src0 = _pad1(src, NW * epw0, jnp.int32)
    dst0 = _pad1(dst, NW * epw0, jnp.int32)
    ew0 = _pad1(ew, NW * epw0, jnp.float32)

    loop = jnp.arange(N, dtype=jnp.int32)
    E1 = E + N
    epw1 = -(-E1 // (NW * B)) * B
    nblk1 = epw1 // B
    src1 = _pad1(jnp.concatenate([src, loop]), NW * epw1, jnp.int32)
    dst1 = _pad1(jnp.concatenate([dst, loop]), NW * epw1, jnp.int32)
    ew1 = _pad1(jnp.concatenate([ew, jnp.ones((N,), jnp.float32)]),
                NW * epw1, jnp.float32)

    xp = jnp.concatenate([x, jnp.zeros((NP - N, x.shape[1]), jnp.float32)])
    znd = jnp.zeros((NP, D), jnp.float32)

    mm = pl.pallas_call(
        _tc_mm_body, grid=(_GRID,),
        in_specs=[_row_spec((D,)), _full_spec((D, D))],
        out_specs=_row_spec((D,)),
        out_shape=_sds((NP, D)))

    hlin0 = mm(xp, W0)
    m0p, degp = _edge0_call(nblk0, hlin0, src0, dst0, ew0, znd)

    dinv = pl.pallas_call(
        _tc_deg_body, grid=(1,),
        in_specs=[pl.BlockSpec((NW, NP), lambda i: (0, 0))],
        out_specs=_full_spec((NP,)),
        out_shape=_sds((NP,)))(degp.reshape(NW, NP))

    comb = pl.pallas_call(
        _tc_comb_body, grid=(_GRID,),
        in_specs=[_parts_spec(), _full_spec((D,)), _full_spec((D, D))],
        out_specs=[_row_spec((D,)), _row_spec((D,))],
        out_shape=[_sds((NP, D)), _sds((NP, D))])

    h1, hlin1 = comb(m0p, b0, W1)
    m1p = _edge_norm_call(nblk1, hlin1, src1, dst1, ew1, dinv, znd)
    h2, hlin2 = comb(m1p, b1, W2)
    m2p = _edge_norm_call(nblk1, hlin2, src1, dst1, ew1, dinv, znd)

    out = pl.pallas_call(
        _tc_final_body, grid=(_GRID,),
        in_specs=[_parts_spec(), _full_spec((D,)),
                  _row_spec((D,)), _row_spec((D,)),
                  _full_spec((3 * D, D)), _full_spec((D,))],
        out_specs=_row_spec((D,)),
        out_shape=_sds((NP, D)),
    )(m2p, b2, h1, h2, Wh, bh)
    return out[0:N]
